# Initial kernel scaffold; baseline (speedup 1.0000x reference)
import functools

import jax
import jax.numpy as jnp
from jax import lax
from jax.experimental import pallas as pl
from jax.experimental.pallas import tpu as pltpu
from jax.experimental.pallas import tpu_sc as plsc

N = 10000
K = 32
D = 128
TOTAL = N * K
NC = 2
NS = 16
NW = NC * NS
PER_W = TOTAL // NW
CH = 80
NCH = PER_W // CH


def _gather_body(table_hbm, idx_hbm, out_hbm, idx_v, rows_v, gsem):
    wid = lax.axis_index("s") * NC + lax.axis_index("c")
    base = wid * PER_W
    pltpu.sync_copy(idx_hbm.at[pl.ds(wid * NCH, NCH)], idx_v)

    def body(j, carry):
        pltpu.async_copy(table_hbm.at[idx_v.at[j]], rows_v, gsem).wait()
        pltpu.sync_copy(rows_v, out_hbm.at[pl.ds(base + j * CH, CH)])
        return carry

    lax.fori_loop(0, NCH, body, 0)


@jax.jit
def kernel(x_features, x_graph):
    table = x_features.reshape(N, D)
    idx = x_graph.reshape(TOTAL // CH, CH)
    mesh = plsc.VectorSubcoreMesh(
        core_axis_name="c", subcore_axis_name="s", num_cores=NC, num_subcores=NS
    )
    out = pl.kernel(
        _gather_body,
        out_type=jax.ShapeDtypeStruct((TOTAL, D), jnp.float32),
        mesh=mesh,
        scratch_types=[
            pltpu.VMEM((NCH, CH), jnp.int32),
            pltpu.VMEM((CH, D), jnp.float32),
            pltpu.SemaphoreType.DMA,
        ],
    )(table, idx)
    return out.reshape(1, N, K, D)


# speedup vs baseline: 3.3370x; 3.3370x over previous
import functools

import jax
import jax.numpy as jnp
from jax import lax
from jax.experimental import pallas as pl
from jax.experimental.pallas import tpu as pltpu
from jax.experimental.pallas import tpu_sc as plsc

N = 10000
K = 32
D = 128
TOTAL = N * K
NC = 2
NS = 16
NW = NC * NS
PER_W = TOTAL // NW
CH = 80
NCH = PER_W // CH


def _gather_body(table_hbm, idx_hbm, out_hbm, idx_v, rows_v, gsem):
    wid = lax.axis_index("s") * NC + lax.axis_index("c")
    base = wid * PER_W
    pltpu.sync_copy(idx_hbm.at[wid], idx_v)

    def body(j, carry):
        pltpu.async_copy(table_hbm.at[idx_v.at[j]], rows_v, gsem).wait()
        pltpu.sync_copy(rows_v, out_hbm.at[pl.ds(base + j * CH, CH)])
        return carry

    lax.fori_loop(0, NCH, body, 0)


@jax.jit
def kernel(x_features, x_graph):
    table = x_features.reshape(N, D)
    idx = x_graph.reshape(NW, NCH, CH)
    mesh = plsc.VectorSubcoreMesh(
        core_axis_name="c", subcore_axis_name="s", num_cores=NC, num_subcores=NS
    )
    out = pl.kernel(
        _gather_body,
        out_type=jax.ShapeDtypeStruct((TOTAL, D), jnp.float32),
        mesh=mesh,
        scratch_types=[
            pltpu.VMEM((NCH, CH), jnp.int32),
            pltpu.VMEM((CH, D), jnp.float32),
            pltpu.SemaphoreType.DMA,
        ],
    )(table, idx)
    return out.reshape(1, N, K, D)


# speedup vs baseline: 5.4381x; 1.6296x over previous
import functools

import jax
import jax.numpy as jnp
from jax import lax
from jax.experimental import pallas as pl
from jax.experimental.pallas import tpu as pltpu
from jax.experimental.pallas import tpu_sc as plsc

N = 10000
K = 32
D = 128
TOTAL = N * K
NC = 2
NS = 16
NW = NC * NS
PER_W = TOTAL // NW
CH = 80
NCH = PER_W // CH
NBUF = 5
NGRP = NCH // NBUF


def _gather_body(table_hbm, idx_hbm, out_hbm, idx_v, rows_v, *sems):
    gsems = sems[:NBUF]
    osems = sems[NBUF:]
    wid = lax.axis_index("s") * NC + lax.axis_index("c")
    base = wid * PER_W
    pltpu.sync_copy(idx_hbm.at[wid], idx_v)

    def g_desc(j, b):
        return pltpu.make_async_copy(
            table_hbm.at[idx_v.at[j]], rows_v.at[b], gsems[b]
        )

    def o_desc(j, b):
        return pltpu.make_async_copy(
            rows_v.at[b], out_hbm.at[pl.ds(base + j * CH, CH)], osems[b]
        )

    for b in range(NBUF):
        g_desc(b, b).start()

    def outer(g, carry):
        j0 = g * NBUF
        for b in range(NBUF):
            g_desc(j0 + b, b).wait()
            o_desc(j0 + b, b).start()
        for b in range(NBUF):
            nxt = j0 + b + NBUF

            @pl.when(nxt < NCH)
            def _():
                o_desc(j0 + b, b).wait()
                g_desc(nxt, b).start()
        return carry

    lax.fori_loop(0, NGRP, outer, 0)
    for b in range(NBUF):
        o_desc(NCH - NBUF + b, b).wait()


@jax.jit
def kernel(x_features, x_graph):
    table = x_features.reshape(N, D)
    idx = x_graph.reshape(NW, NCH, CH)
    mesh = plsc.VectorSubcoreMesh(
        core_axis_name="c", subcore_axis_name="s", num_cores=NC, num_subcores=NS
    )
    out = pl.kernel(
        _gather_body,
        out_type=jax.ShapeDtypeStruct((TOTAL, D), jnp.float32),
        mesh=mesh,
        scratch_types=(
            [
                pltpu.VMEM((NCH, CH), jnp.int32),
                pltpu.VMEM((NBUF, CH, D), jnp.float32),
            ]
            + [pltpu.SemaphoreType.DMA] * (2 * NBUF)
        ),
    )(table, idx)
    return out.reshape(1, N, K, D)


# speedup vs baseline: 5.5224x; 1.0155x over previous
import functools

import jax
import jax.numpy as jnp
from jax import lax
from jax.experimental import pallas as pl
from jax.experimental.pallas import tpu as pltpu
from jax.experimental.pallas import tpu_sc as plsc

N = 10000
K = 32
D = 128
TOTAL = N * K
NC = 2
NS = 16
NW = NC * NS
PER_W = TOTAL // NW
CH = 80
NCH = PER_W // CH
NBUF = 10
NGRP = -(-NCH // NBUF)


def _gather_body(table_hbm, idx_hbm, out_hbm, idx_v, rows_v, *sems):
    gsems = sems[:NBUF]
    osems = sems[NBUF:]
    wid = lax.axis_index("s") * NC + lax.axis_index("c")
    base = wid * PER_W
    pltpu.sync_copy(idx_hbm.at[wid], idx_v)

    def g_desc(j, b):
        return pltpu.make_async_copy(
            table_hbm.at[idx_v.at[j]], rows_v.at[b], gsems[b]
        )

    def o_desc(j, b):
        return pltpu.make_async_copy(
            rows_v.at[b], out_hbm.at[pl.ds(base + j * CH, CH)], osems[b]
        )

    for b in range(NBUF):
        g_desc(b, b).start()

    def outer(g, carry):
        j0 = g * NBUF
        for b in range(NBUF):
            j = j0 + b

            @pl.when(j < NCH)
            def _():
                g_desc(j, b).wait()
                o_desc(j, b).start()

        for b in range(NBUF):
            nxt = j0 + b + NBUF

            @pl.when(nxt < NCH)
            def _():
                o_desc(j0 + b, b).wait()
                g_desc(nxt, b).start()
        return carry

    lax.fori_loop(0, NGRP, outer, 0)
    for j in range(NCH - NBUF, NCH):
        o_desc(j, j % NBUF).wait()


@jax.jit
def kernel(x_features, x_graph):
    table = x_features.reshape(N, D)
    idx = x_graph.reshape(NW, NCH, CH)
    mesh = plsc.VectorSubcoreMesh(
        core_axis_name="c", subcore_axis_name="s", num_cores=NC, num_subcores=NS
    )
    out = pl.kernel(
        _gather_body,
        out_type=jax.ShapeDtypeStruct((TOTAL, D), jnp.float32),
        mesh=mesh,
        scratch_types=(
            [
                pltpu.VMEM((NCH, CH), jnp.int32),
                pltpu.VMEM((NBUF, CH, D), jnp.float32),
            ]
            + [pltpu.SemaphoreType.DMA] * (2 * NBUF)
        ),
    )(table, idx)
    return out.reshape(1, N, K, D)
